# Initial kernel scaffold; baseline (speedup 1.0000x reference)
import jax
import jax.numpy as jnp
from jax.experimental import pallas as pl


def kernel(species, coordinates, conv_tensor):
    raise NotImplementedError("write your pallas kernel here")


# speedup vs baseline: 238.5528x; 238.5528x over previous
import functools

import jax
import jax.numpy as jnp
from jax import lax
from jax.experimental import pallas as pl
from jax.experimental.pallas import tpu as pltpu
from jax.experimental.pallas import tpu_sc as plsc

_N = 16384 * 200
_NC, _NS, _L = 2, 16, 16
_NW = _NC * _NS
_PER_W = _N // _NW
_CH = 12800
_NCHUNK = _PER_W // _CH
_TAB = 128


def _make_sc_gather():
    mesh = plsc.VectorSubcoreMesh(core_axis_name="c", subcore_axis_name="s")

    @functools.partial(
        pl.kernel,
        out_type=jax.ShapeDtypeStruct((_N,), jnp.int32),
        mesh=mesh,
        scratch_types=[
            pltpu.VMEM((_TAB,), jnp.int32),
            pltpu.VMEM((_CH,), jnp.int32),
            pltpu.VMEM((_CH,), jnp.int32),
            pltpu.VMEM((_CH,), jnp.int32),
            pltpu.VMEM((_CH,), jnp.int32),
            pltpu.SemaphoreType.DMA,
            pltpu.SemaphoreType.DMA,
            pltpu.SemaphoreType.DMA,
            pltpu.SemaphoreType.DMA,
        ],
        compiler_params=pltpu.CompilerParams(needs_layout_passes=False),
    )
    def conv_gather(spec_hbm, conv_hbm, out_hbm, tab_v, in_v0, in_v1,
                    out_v0, out_v1, in_sem0, in_sem1, out_sem0, out_sem1):
        wid = lax.axis_index("s") * _NC + lax.axis_index("c")
        base = wid * _PER_W
        pltpu.sync_copy(conv_hbm, tab_v)
        in_bufs = (in_v0, in_v1)
        out_bufs = (out_v0, out_v1)
        in_sems = (in_sem0, in_sem1)
        out_sems = (out_sem0, out_sem1)
        in_cp = [None] * _NCHUNK
        out_cp = [None] * _NCHUNK
        in_cp[0] = pltpu.async_copy(
            spec_hbm.at[pl.ds(base, _CH)], in_bufs[0], in_sems[0])
        for c in range(_NCHUNK):
            b = c % 2
            if c + 1 < _NCHUNK:
                in_cp[c + 1] = pltpu.async_copy(
                    spec_hbm.at[pl.ds(base + (c + 1) * _CH, _CH)],
                    in_bufs[1 - b], in_sems[1 - b])
            in_cp[c].wait()
            if c >= 2:
                out_cp[c - 2].wait()
            in_b, out_b = in_bufs[b], out_bufs[b]

            @plsc.parallel_loop(0, _CH, _L, unroll=8)
            def _body(i):
                idx = in_b[pl.ds(i, _L)]
                out_b[pl.ds(i, _L)] = plsc.load_gather(tab_v, [idx])

            out_cp[c] = pltpu.async_copy(
                out_b, out_hbm.at[pl.ds(base + c * _CH, _CH)],
                out_sems[b])
        out_cp[_NCHUNK - 2].wait()
        out_cp[_NCHUNK - 1].wait()

    return conv_gather


def kernel(species, coordinates, conv_tensor):
    conv_pad = jnp.pad(conv_tensor, (0, _TAB - conv_tensor.shape[0]))
    flat = species.reshape(-1)
    out = _make_sc_gather()(flat, conv_pad)
    return (out.reshape(species.shape), coordinates)
